# baseline (speedup 1.0000x reference)
import functools

import jax
import jax.numpy as jnp
from jax.experimental import pallas as pl
from jax.experimental.pallas import tpu as pltpu

_EPS = 1e-5
_VMEM_LIMIT = 56 * 1024 * 1024
_STRIP_BYTES_MAX = 22 * 1024 * 1024


_N_SUB = 4
_N_CHUNKS = 4


def _bn_manual_kernel(x_hbm, g_ref, b_ref, y_hbm, xbuf, in_sem, out_sem, *,
                      cols, sub_cols, n_sub, chunk, n_chunks, inv_n, eps):
    col0 = pl.program_id(0) * cols

    def in_copy(s, k):
        return pltpu.make_async_copy(
            x_hbm.at[pl.ds(k * chunk, chunk), pl.ds(col0 + s * sub_cols, sub_cols)],
            xbuf.at[pl.ds(k * chunk, chunk), pl.ds(s * sub_cols, sub_cols)],
            in_sem.at[s * n_chunks + k])

    def out_copy(s, k):
        return pltpu.make_async_copy(
            xbuf.at[pl.ds(k * chunk, chunk), pl.ds(s * sub_cols, sub_cols)],
            y_hbm.at[pl.ds(k * chunk, chunk), pl.ds(col0 + s * sub_cols, sub_cols)],
            out_sem.at[s * n_chunks + k])

    for s in range(n_sub):
        for k in range(n_chunks):
            in_copy(s, k).start()

    inv = jnp.float32(inv_n)
    for s in range(n_sub):
        cs = pl.ds(s * sub_cols, sub_cols)
        acc = jnp.zeros((1, sub_cols), jnp.float32)
        acc2 = jnp.zeros((1, sub_cols), jnp.float32)
        for k in range(n_chunks):
            in_copy(s, k).wait()
            xk = xbuf[pl.ds(k * chunk, chunk), cs].astype(jnp.float32)
            acc = acc + jnp.sum(xk, axis=0, keepdims=True)
            acc2 = acc2 + jnp.sum(xk * xk, axis=0, keepdims=True)
        m1 = acc * inv
        var = jnp.maximum(acc2 * inv - m1 * m1, 0.0)
        scale = g_ref[0:1, cs] * jax.lax.rsqrt(var + eps)
        shift = b_ref[0:1, cs] - m1 * scale
        for k in range(n_chunks):
            rs = pl.ds(k * chunk, chunk)
            xbuf[rs, cs] = xbuf[rs, cs] * scale + shift
            out_copy(s, k).start()
    for s in range(n_sub):
        for k in range(n_chunks):
            out_copy(s, k).wait()


def _bn_manual(x, g2d, b2d, eps):
    n, c = x.shape
    cols = c // 2
    n_sub = _N_SUB if cols % (_N_SUB * 128) == 0 else 1
    sub_cols = cols // n_sub
    n_chunks = _N_CHUNKS if n % (_N_CHUNKS * 8) == 0 else 1
    chunk = n // n_chunks
    body = functools.partial(
        _bn_manual_kernel,
        cols=cols, sub_cols=sub_cols, n_sub=n_sub,
        chunk=chunk, n_chunks=n_chunks, inv_n=1.0 / n, eps=eps)
    return pl.pallas_call(
        body,
        out_shape=jax.ShapeDtypeStruct((n, c), x.dtype),
        grid=(2,),
        in_specs=[
            pl.BlockSpec(memory_space=pl.ANY),
            pl.BlockSpec((1, cols), lambda j: (0, j)),
            pl.BlockSpec((1, cols), lambda j: (0, j)),
        ],
        out_specs=pl.BlockSpec(memory_space=pl.ANY),
        scratch_shapes=[
            pltpu.VMEM((n, cols), x.dtype),
            pltpu.SemaphoreType.DMA((n_sub * n_chunks,)),
            pltpu.SemaphoreType.DMA((n_sub * n_chunks,)),
        ],
        compiler_params=pltpu.CompilerParams(
            dimension_semantics=("parallel",),
            vmem_limit_bytes=_VMEM_LIMIT,
        ),
    )(x, g2d, b2d)


def _bn_strip_kernel(x_ref, g_ref, b_ref, y_ref, *, inv_n, eps):
    x = x_ref[...].astype(jnp.float32)
    inv = jnp.float32(inv_n)
    m1 = jnp.sum(x, axis=0, keepdims=True) * inv
    m2 = jnp.sum(x * x, axis=0, keepdims=True) * inv
    var = jnp.maximum(m2 - m1 * m1, 0.0)
    k = g_ref[...] * jax.lax.rsqrt(var + eps)
    y_ref[...] = ((x - m1) * k + b_ref[...]).astype(y_ref.dtype)


def _bn_strips(x, g2d, b2d, eps):
    n, c = x.shape
    if c % 256 == 0 and n * 256 * x.dtype.itemsize <= 8 * 1024 * 1024:
        tile_c = 256
    elif c % 128 == 0 and n * 128 * x.dtype.itemsize <= 8 * 1024 * 1024:
        tile_c = 128
    else:
        tile_c = c
    body = functools.partial(_bn_strip_kernel, inv_n=1.0 / n, eps=eps)
    return pl.pallas_call(
        body,
        out_shape=jax.ShapeDtypeStruct((n, c), x.dtype),
        grid=(c // tile_c,),
        in_specs=[
            pl.BlockSpec((n, tile_c), lambda j: (0, j)),
            pl.BlockSpec((1, tile_c), lambda j: (0, j)),
            pl.BlockSpec((1, tile_c), lambda j: (0, j)),
        ],
        out_specs=pl.BlockSpec((n, tile_c), lambda j: (0, j)),
        compiler_params=pltpu.CompilerParams(
            dimension_semantics=("parallel",),
            vmem_limit_bytes=_VMEM_LIMIT,
        ),
    )(x, g2d, b2d)


def kernel(x, gamma, beta):
    n, c = x.shape
    g2d = gamma.astype(jnp.float32).reshape(1, c)
    b2d = beta.astype(jnp.float32).reshape(1, c)
    strip_ok = (
        c % 256 == 0
        and n % 8 == 0
        and n * (c // 2) * x.dtype.itemsize <= _STRIP_BYTES_MAX
    )
    if strip_ok:
        return _bn_manual(x, g2d, b2d, _EPS)
    return _bn_strips(x, g2d, b2d, _EPS)


# speedup vs baseline: 1.0377x; 1.0377x over previous
import functools

import jax
import jax.numpy as jnp
from jax.experimental import pallas as pl
from jax.experimental.pallas import tpu as pltpu

_EPS = 1e-5
_VMEM_LIMIT = 56 * 1024 * 1024
_STRIP_BYTES_MAX = 22 * 1024 * 1024


_N_SUB = 2
_N_CHUNKS = 8


def _bn_manual_kernel(x_hbm, g_ref, b_ref, y_hbm, xbuf, in_sem, out_sem, *,
                      cols, sub_cols, n_sub, chunk, n_chunks, inv_n, eps):
    col0 = pl.program_id(0) * cols

    def in_copy(s, k):
        return pltpu.make_async_copy(
            x_hbm.at[pl.ds(k * chunk, chunk), pl.ds(col0 + s * sub_cols, sub_cols)],
            xbuf.at[pl.ds(k * chunk, chunk), pl.ds(s * sub_cols, sub_cols)],
            in_sem.at[s * n_chunks + k])

    def out_copy(s, k):
        return pltpu.make_async_copy(
            xbuf.at[pl.ds(k * chunk, chunk), pl.ds(s * sub_cols, sub_cols)],
            y_hbm.at[pl.ds(k * chunk, chunk), pl.ds(col0 + s * sub_cols, sub_cols)],
            out_sem.at[s * n_chunks + k])

    for s in range(n_sub):
        for k in range(n_chunks):
            in_copy(s, k).start()

    inv = jnp.float32(inv_n)
    for s in range(n_sub):
        cs = pl.ds(s * sub_cols, sub_cols)
        acc = jnp.zeros((1, sub_cols), jnp.float32)
        acc2 = jnp.zeros((1, sub_cols), jnp.float32)
        for k in range(n_chunks):
            in_copy(s, k).wait()
            xk = xbuf[pl.ds(k * chunk, chunk), cs].astype(jnp.float32)
            acc = acc + jnp.sum(xk, axis=0, keepdims=True)
            acc2 = acc2 + jnp.sum(xk * xk, axis=0, keepdims=True)
        m1 = acc * inv
        var = jnp.maximum(acc2 * inv - m1 * m1, 0.0)
        scale = g_ref[0:1, cs] * jax.lax.rsqrt(var + eps)
        shift = b_ref[0:1, cs] - m1 * scale
        for k in range(n_chunks):
            rs = pl.ds(k * chunk, chunk)
            xbuf[rs, cs] = xbuf[rs, cs] * scale + shift
            out_copy(s, k).start()
    for s in range(n_sub):
        for k in range(n_chunks):
            out_copy(s, k).wait()


def _bn_manual(x, g2d, b2d, eps):
    n, c = x.shape
    cols = c // 2
    n_sub = _N_SUB if cols % (_N_SUB * 128) == 0 else 1
    sub_cols = cols // n_sub
    n_chunks = _N_CHUNKS if n % (_N_CHUNKS * 8) == 0 else 1
    chunk = n // n_chunks
    body = functools.partial(
        _bn_manual_kernel,
        cols=cols, sub_cols=sub_cols, n_sub=n_sub,
        chunk=chunk, n_chunks=n_chunks, inv_n=1.0 / n, eps=eps)
    return pl.pallas_call(
        body,
        out_shape=jax.ShapeDtypeStruct((n, c), x.dtype),
        grid=(2,),
        in_specs=[
            pl.BlockSpec(memory_space=pl.ANY),
            pl.BlockSpec((1, cols), lambda j: (0, j)),
            pl.BlockSpec((1, cols), lambda j: (0, j)),
        ],
        out_specs=pl.BlockSpec(memory_space=pl.ANY),
        scratch_shapes=[
            pltpu.VMEM((n, cols), x.dtype),
            pltpu.SemaphoreType.DMA((n_sub * n_chunks,)),
            pltpu.SemaphoreType.DMA((n_sub * n_chunks,)),
        ],
        compiler_params=pltpu.CompilerParams(
            dimension_semantics=("parallel",),
            vmem_limit_bytes=_VMEM_LIMIT,
        ),
    )(x, g2d, b2d)


def _bn_strip_kernel(x_ref, g_ref, b_ref, y_ref, *, inv_n, eps):
    x = x_ref[...].astype(jnp.float32)
    inv = jnp.float32(inv_n)
    m1 = jnp.sum(x, axis=0, keepdims=True) * inv
    m2 = jnp.sum(x * x, axis=0, keepdims=True) * inv
    var = jnp.maximum(m2 - m1 * m1, 0.0)
    k = g_ref[...] * jax.lax.rsqrt(var + eps)
    y_ref[...] = ((x - m1) * k + b_ref[...]).astype(y_ref.dtype)


def _bn_strips(x, g2d, b2d, eps):
    n, c = x.shape
    if c % 256 == 0 and n * 256 * x.dtype.itemsize <= 8 * 1024 * 1024:
        tile_c = 256
    elif c % 128 == 0 and n * 128 * x.dtype.itemsize <= 8 * 1024 * 1024:
        tile_c = 128
    else:
        tile_c = c
    body = functools.partial(_bn_strip_kernel, inv_n=1.0 / n, eps=eps)
    return pl.pallas_call(
        body,
        out_shape=jax.ShapeDtypeStruct((n, c), x.dtype),
        grid=(c // tile_c,),
        in_specs=[
            pl.BlockSpec((n, tile_c), lambda j: (0, j)),
            pl.BlockSpec((1, tile_c), lambda j: (0, j)),
            pl.BlockSpec((1, tile_c), lambda j: (0, j)),
        ],
        out_specs=pl.BlockSpec((n, tile_c), lambda j: (0, j)),
        compiler_params=pltpu.CompilerParams(
            dimension_semantics=("parallel",),
            vmem_limit_bytes=_VMEM_LIMIT,
        ),
    )(x, g2d, b2d)


def kernel(x, gamma, beta):
    n, c = x.shape
    g2d = gamma.astype(jnp.float32).reshape(1, c)
    b2d = beta.astype(jnp.float32).reshape(1, c)
    strip_ok = (
        c % 256 == 0
        and n % 8 == 0
        and n * (c // 2) * x.dtype.itemsize <= _STRIP_BYTES_MAX
    )
    if strip_ok:
        return _bn_manual(x, g2d, b2d, _EPS)
    return _bn_strips(x, g2d, b2d, _EPS)
